# baseline (speedup 1.0000x reference)
import functools

import jax
import jax.numpy as jnp
from jax import lax
from jax.experimental import pallas as pl
from jax.experimental.pallas import tpu as pltpu
from jax.experimental.pallas import tpu_sc as plsc

E = 800000
EMB_DIM = 64
REL_DIM = 16
N_REL = 1000
N_REL_PAD = 1024

NC = 2
NS = 16
NW = NC * NS
L = 16

NPART = 5
EPART = E // NPART
B_PER_W = EPART // NW
CHUNK = 256
FULL_CHUNKS = B_PER_W // CHUNK
TAIL_E = B_PER_W - FULL_CHUNKS * CHUNK


def _project_body(t_ref, w_ref, bias_ref, o_ref):
    wr = w_ref[EMB_DIM:EMB_DIM + REL_DIM, :]
    proj = jnp.dot(t_ref[...], wr, preferred_element_type=jnp.float32)
    proj = proj + bias_ref[...]
    o_ref[...] = jnp.concatenate(
        [proj, jnp.zeros((N_REL_PAD - N_REL, EMB_DIM), jnp.float32)], axis=0)


def _project_table(rel_table, W, bias2d):
    in_dim = 2 * EMB_DIM + REL_DIM
    return pl.pallas_call(
        _project_body,
        in_specs=[
            pl.BlockSpec((N_REL, REL_DIM), lambda: (0, 0)),
            pl.BlockSpec((in_dim, EMB_DIM), lambda: (0, 0)),
            pl.BlockSpec((1, EMB_DIM), lambda: (0, 0)),
        ],
        out_specs=pl.BlockSpec((N_REL_PAD, EMB_DIM), lambda: (0, 0)),
        out_shape=jax.ShapeDtypeStruct((N_REL_PAD, EMB_DIM), jnp.float32),
    )(rel_table, W, bias2d)


def _sc_gather(table_pad, rel_ids, part):
    mesh = plsc.VectorSubcoreMesh(core_axis_name="c", subcore_axis_name="s")

    @functools.partial(
        pl.kernel,
        mesh=mesh,
        out_type=jax.ShapeDtypeStruct((EPART, EMB_DIM), jnp.float32),
        compiler_params=pltpu.CompilerParams(needs_layout_passes=False),
        scratch_types=[
            pltpu.VMEM((N_REL_PAD * EMB_DIM,), jnp.float32),
            pltpu.VMEM((B_PER_W,), jnp.int32),
            pltpu.VMEM((CHUNK, EMB_DIM), jnp.float32),
        ],
    )
    def k(table_hbm, idx_hbm, out_hbm, table_v, idx_v, stg):
        wid = lax.axis_index("s") * NC + lax.axis_index("c")
        base = wid * B_PER_W
        pltpu.sync_copy(table_hbm, table_v)
        pltpu.sync_copy(idx_hbm.at[pl.ds(part * EPART + base, B_PER_W)], idx_v)
        lanes = lax.iota(jnp.int32, L)
        zero16 = lanes * 0

        def gather_edge(goff, row):
            ids16 = plsc.load_gather(idx_v, [zero16 + goff])
            tbase = ids16 * EMB_DIM
            for kq in range(EMB_DIM // L):
                v = plsc.load_gather(table_v, [tbase + (lanes + kq * L)])
                stg[row, pl.ds(kq * L, L)] = v

        @pl.loop(0, FULL_CHUNKS)
        def _(c):
            @plsc.parallel_loop(0, CHUNK, unroll=8)
            def _(e):
                gather_edge(c * CHUNK + e, e)
            pltpu.sync_copy(stg, out_hbm.at[pl.ds(base + c * CHUNK, CHUNK)])

        toff = FULL_CHUNKS * CHUNK

        @plsc.parallel_loop(0, TAIL_E, unroll=8)
        def _(e):
            gather_edge(toff + e, e)

        pltpu.sync_copy(stg.at[pl.ds(0, TAIL_E)],
                        out_hbm.at[pl.ds(base + toff, TAIL_E)])

    return k(table_pad, rel_ids)


def _tc_part(emb_a, rel_part, emb_b, W, prev, part):
    in_dim = 2 * EMB_DIM + REL_DIM
    nblk = EPART // BE
    off = part * nblk

    in_specs = [
        pl.BlockSpec((BE, EMB_DIM), lambda i: (i + off, 0)),
        pl.BlockSpec((BE, EMB_DIM), lambda i: (i, 0)),
        pl.BlockSpec((BE, EMB_DIM), lambda i: (i + off, 0)),
        pl.BlockSpec((in_dim, EMB_DIM), lambda i: (0, 0)),
    ]
    args = [emb_a, rel_part, emb_b, W]
    aliases = {}
    if prev is None:
        body = _tc_body
    else:
        def body(a_ref, rel_ref, b2_ref, w_ref, prev_ref, o_ref):
            del prev_ref
            _tc_body(a_ref, rel_ref, b2_ref, w_ref, o_ref)

        in_specs.append(pl.BlockSpec(memory_space=pltpu.MemorySpace.HBM))
        args.append(prev)
        aliases = {4: 0}
    return pl.pallas_call(
        body,
        grid=(nblk,),
        in_specs=in_specs,
        out_specs=pl.BlockSpec((BE, EMB_DIM), lambda i: (i + off, 0)),
        out_shape=jax.ShapeDtypeStruct((E, EMB_DIM), jnp.float32),
        input_output_aliases=aliases,
        compiler_params=pltpu.CompilerParams(
            dimension_semantics=("parallel",),
        ),
    )(*args)


BE = 8000


def _tc_body(a_ref, rel_ref, b2_ref, w_ref, o_ref):
    wa = w_ref[0:EMB_DIM, :]
    wb = w_ref[EMB_DIM + REL_DIM:, :]
    acc = jnp.dot(a_ref[...], wa, preferred_element_type=jnp.float32)
    acc += jnp.dot(b2_ref[...], wb, preferred_element_type=jnp.float32)
    o_ref[...] = acc + rel_ref[...]


def kernel(emb_a, rel_ids, emb_b, rel_table, W, b):
    table_pad = _project_table(rel_table, W, b.reshape(1, EMB_DIM))
    table_flat = table_pad.reshape(N_REL_PAD * EMB_DIM)
    rels = [_sc_gather(table_flat, rel_ids, part) for part in range(NPART)]
    out = None
    for part in range(NPART):
        out = _tc_part(emb_a, rels[part], emb_b, W, out, part)
    return out


# speedup vs baseline: 1.0279x; 1.0279x over previous
import functools

import jax
import jax.numpy as jnp
from jax import lax
from jax.experimental import pallas as pl
from jax.experimental.pallas import tpu as pltpu
from jax.experimental.pallas import tpu_sc as plsc

E = 800000
EMB_DIM = 64
REL_DIM = 16
N_REL = 1000
N_REL_PAD = 1024

NC = 2
NS = 16
NW = NC * NS
L = 16

B_TOTAL = E // NW
PHASES = 5
B_PHASE = B_TOTAL // PHASES
CHUNK = 256
FULL_CHUNKS = B_PHASE // CHUNK
TAIL_E = B_PHASE - FULL_CHUNKS * CHUNK


def _project_body(t_ref, w_ref, bias_ref, o_ref):
    wr = w_ref[EMB_DIM:EMB_DIM + REL_DIM, :]
    proj = jnp.dot(t_ref[...], wr, preferred_element_type=jnp.float32)
    proj = proj + bias_ref[...]
    o_ref[...] = jnp.concatenate(
        [proj, jnp.zeros((N_REL_PAD - N_REL, EMB_DIM), jnp.float32)], axis=0)


def _project_table(rel_table, W, bias2d):
    in_dim = 2 * EMB_DIM + REL_DIM
    return pl.pallas_call(
        _project_body,
        in_specs=[
            pl.BlockSpec((N_REL, REL_DIM), lambda: (0, 0)),
            pl.BlockSpec((in_dim, EMB_DIM), lambda: (0, 0)),
            pl.BlockSpec((1, EMB_DIM), lambda: (0, 0)),
        ],
        out_specs=pl.BlockSpec((N_REL_PAD, EMB_DIM), lambda: (0, 0)),
        out_shape=jax.ShapeDtypeStruct((N_REL_PAD, EMB_DIM), jnp.float32),
    )(rel_table, W, bias2d)


def _sc_gather(table_pad, rel_ids):
    mesh = plsc.VectorSubcoreMesh(core_axis_name="c", subcore_axis_name="s")

    @functools.partial(
        pl.kernel,
        mesh=mesh,
        out_type=jax.ShapeDtypeStruct((E, EMB_DIM), jnp.float32),
        compiler_params=pltpu.CompilerParams(needs_layout_passes=False),
        scratch_types=[
            pltpu.VMEM((N_REL_PAD * EMB_DIM,), jnp.float32),
            pltpu.VMEM((B_PHASE,), jnp.int32),
            pltpu.VMEM((CHUNK, EMB_DIM), jnp.float32),
        ],
    )
    def k(table_hbm, idx_hbm, out_hbm, table_v, idx_v, stg):
        wid = lax.axis_index("s") * NC + lax.axis_index("c")
        base = wid * B_TOTAL
        pltpu.sync_copy(table_hbm, table_v)
        lanes = lax.iota(jnp.int32, L)
        zero16 = lanes * 0

        def gather_edge(goff, row):
            ids16 = plsc.load_gather(idx_v, [zero16 + goff])
            tbase = ids16 * EMB_DIM
            for kq in range(EMB_DIM // L):
                v = plsc.load_gather(table_v, [tbase + (lanes + kq * L)])
                stg[row, pl.ds(kq * L, L)] = v

        @pl.loop(0, PHASES)
        def _(p):
            poff = base + p * B_PHASE
            pltpu.sync_copy(idx_hbm.at[pl.ds(poff, B_PHASE)], idx_v)

            @pl.loop(0, FULL_CHUNKS)
            def _(c):
                @plsc.parallel_loop(0, CHUNK, unroll=8)
                def _(e):
                    gather_edge(c * CHUNK + e, e)
                pltpu.sync_copy(stg, out_hbm.at[pl.ds(poff + c * CHUNK, CHUNK)])

            toff = FULL_CHUNKS * CHUNK

            @plsc.parallel_loop(0, TAIL_E, unroll=8)
            def _(e):
                gather_edge(toff + e, e)

            pltpu.sync_copy(stg.at[pl.ds(0, TAIL_E)],
                            out_hbm.at[pl.ds(poff + toff, TAIL_E)])

    return k(table_pad, rel_ids)


def _tc_fused(emb_a, rel_g, emb_b, W):
    in_dim = 2 * EMB_DIM + REL_DIM
    return pl.pallas_call(
        _tc_body,
        grid=(E // BE,),
        in_specs=[
            pl.BlockSpec((BE, EMB_DIM), lambda i: (i, 0)),
            pl.BlockSpec((BE, EMB_DIM), lambda i: (i, 0)),
            pl.BlockSpec((BE, EMB_DIM), lambda i: (i, 0)),
            pl.BlockSpec((in_dim, EMB_DIM), lambda i: (0, 0)),
        ],
        out_specs=pl.BlockSpec((BE, EMB_DIM), lambda i: (i, 0)),
        out_shape=jax.ShapeDtypeStruct((E, EMB_DIM), jnp.float32),
        compiler_params=pltpu.CompilerParams(
            dimension_semantics=("parallel",),
        ),
    )(emb_a, rel_g, emb_b, W)


BE = 8000


def _tc_body(a_ref, rel_ref, b2_ref, w_ref, o_ref):
    wa = w_ref[0:EMB_DIM, :]
    wb = w_ref[EMB_DIM + REL_DIM:, :]
    acc = jnp.dot(a_ref[...], wa, preferred_element_type=jnp.float32)
    acc += jnp.dot(b2_ref[...], wb, preferred_element_type=jnp.float32)
    o_ref[...] = acc + rel_ref[...]


def kernel(emb_a, rel_ids, emb_b, rel_table, W, b):
    table_pad = _project_table(rel_table, W, b.reshape(1, EMB_DIM))
    table_flat = table_pad.reshape(N_REL_PAD * EMB_DIM)
    rel_g = _sc_gather(table_flat, rel_ids)
    return _tc_fused(emb_a, rel_g, emb_b, W)


# speedup vs baseline: 1.0279x; 1.0000x over previous
import functools

import jax
import jax.numpy as jnp
from jax import lax
from jax.experimental import pallas as pl
from jax.experimental.pallas import tpu as pltpu
from jax.experimental.pallas import tpu_sc as plsc

E = 800000
EMB_DIM = 64
REL_DIM = 16
N_REL = 1000
N_REL_PAD = 1024

NC = 2
NS = 16
NW = NC * NS
L = 16

B_TOTAL = E // NW
PHASES = 5
B_PHASE = B_TOTAL // PHASES
CHUNK = 256
FULL_CHUNKS = B_PHASE // CHUNK
TAIL_E = B_PHASE - FULL_CHUNKS * CHUNK


def _project_body(t_ref, w_ref, bias_ref, o_ref):
    wr = w_ref[EMB_DIM:EMB_DIM + REL_DIM, :]
    proj = jnp.dot(t_ref[...], wr, preferred_element_type=jnp.float32)
    proj = proj + bias_ref[...]
    o_ref[...] = jnp.concatenate(
        [proj, jnp.zeros((N_REL_PAD - N_REL, EMB_DIM), jnp.float32)], axis=0)


def _project_table(rel_table, W, bias2d):
    in_dim = 2 * EMB_DIM + REL_DIM
    return pl.pallas_call(
        _project_body,
        in_specs=[
            pl.BlockSpec((N_REL, REL_DIM), lambda: (0, 0)),
            pl.BlockSpec((in_dim, EMB_DIM), lambda: (0, 0)),
            pl.BlockSpec((1, EMB_DIM), lambda: (0, 0)),
        ],
        out_specs=pl.BlockSpec((N_REL_PAD, EMB_DIM), lambda: (0, 0)),
        out_shape=jax.ShapeDtypeStruct((N_REL_PAD, EMB_DIM), jnp.float32),
    )(rel_table, W, bias2d)


_BCAST_DNUMS = lax.GatherDimensionNumbers(
    offset_dims=(), collapsed_slice_dims=(0,), start_index_map=(0,))


def _bcast_lane(vec, j):
    idx = jnp.full((L, 1), j, jnp.int32)
    return lax.gather(vec, idx, _BCAST_DNUMS, (1,),
                      mode=lax.GatherScatterMode.PROMISE_IN_BOUNDS)


def _sc_gather(table_pad, rel_ids):
    mesh = plsc.VectorSubcoreMesh(core_axis_name="c", subcore_axis_name="s")

    @functools.partial(
        pl.kernel,
        mesh=mesh,
        out_type=jax.ShapeDtypeStruct((E, EMB_DIM), jnp.float32),
        compiler_params=pltpu.CompilerParams(needs_layout_passes=False),
        scratch_types=[
            pltpu.VMEM((N_REL_PAD * EMB_DIM,), jnp.float32),
            pltpu.VMEM((B_PHASE,), jnp.int32),
            pltpu.VMEM((CHUNK, EMB_DIM), jnp.float32),
        ],
    )
    def k(table_hbm, idx_hbm, out_hbm, table_v, idx_v, stg):
        wid = lax.axis_index("s") * NC + lax.axis_index("c")
        base = wid * B_TOTAL
        pltpu.sync_copy(table_hbm, table_v)
        lanes = lax.iota(jnp.int32, L)
        zero16 = lanes * 0

        def gather_edge(goff, row):
            ids16 = plsc.load_gather(idx_v, [zero16 + goff])
            tbase = ids16 * EMB_DIM
            for kq in range(EMB_DIM // L):
                v = plsc.load_gather(table_v, [tbase + (lanes + kq * L)])
                stg[row, pl.ds(kq * L, L)] = v

        @pl.loop(0, PHASES)
        def _(p):
            poff = base + p * B_PHASE
            pltpu.sync_copy(idx_hbm.at[pl.ds(poff, B_PHASE)], idx_v)

            @pl.loop(0, FULL_CHUNKS)
            def _(c):
                @plsc.parallel_loop(0, CHUNK // L, unroll=2)
                def _(gg):
                    gvec = idx_v[pl.ds(c * CHUNK + gg * L, L)]
                    for j in range(L):
                        idsj = _bcast_lane(gvec, j)
                        tbase = idsj * EMB_DIM
                        for kq in range(EMB_DIM // L):
                            v = plsc.load_gather(
                                table_v, [tbase + (lanes + kq * L)])
                            stg[gg * L + j, pl.ds(kq * L, L)] = v
                pltpu.sync_copy(stg, out_hbm.at[pl.ds(poff + c * CHUNK, CHUNK)])

            toff = FULL_CHUNKS * CHUNK

            @plsc.parallel_loop(0, TAIL_E, unroll=8)
            def _(e):
                gather_edge(toff + e, e)

            pltpu.sync_copy(stg.at[pl.ds(0, TAIL_E)],
                            out_hbm.at[pl.ds(poff + toff, TAIL_E)])

    return k(table_pad, rel_ids)


def _tc_fused(emb_a, rel_g, emb_b, W):
    in_dim = 2 * EMB_DIM + REL_DIM
    return pl.pallas_call(
        _tc_body,
        grid=(E // BE,),
        in_specs=[
            pl.BlockSpec((BE, EMB_DIM), lambda i: (i, 0)),
            pl.BlockSpec((BE, EMB_DIM), lambda i: (i, 0)),
            pl.BlockSpec((BE, EMB_DIM), lambda i: (i, 0)),
            pl.BlockSpec((in_dim, EMB_DIM), lambda i: (0, 0)),
        ],
        out_specs=pl.BlockSpec((BE, EMB_DIM), lambda i: (i, 0)),
        out_shape=jax.ShapeDtypeStruct((E, EMB_DIM), jnp.float32),
        compiler_params=pltpu.CompilerParams(
            dimension_semantics=("parallel",),
        ),
    )(emb_a, rel_g, emb_b, W)


BE = 8000


def _tc_body(a_ref, rel_ref, b2_ref, w_ref, o_ref):
    wa = w_ref[0:EMB_DIM, :]
    wb = w_ref[EMB_DIM + REL_DIM:, :]
    acc = jnp.dot(a_ref[...], wa, preferred_element_type=jnp.float32)
    acc += jnp.dot(b2_ref[...], wb, preferred_element_type=jnp.float32)
    o_ref[...] = acc + rel_ref[...]


def kernel(emb_a, rel_ids, emb_b, rel_table, W, b):
    table_pad = _project_table(rel_table, W, b.reshape(1, EMB_DIM))
    table_flat = table_pad.reshape(N_REL_PAD * EMB_DIM)
    rel_g = _sc_gather(table_flat, rel_ids)
    return _tc_fused(emb_a, rel_g, emb_b, W)


# speedup vs baseline: 1.0289x; 1.0010x over previous
import functools

import jax
import jax.numpy as jnp
from jax import lax
from jax.experimental import pallas as pl
from jax.experimental.pallas import tpu as pltpu
from jax.experimental.pallas import tpu_sc as plsc

E = 800000
EMB_DIM = 64
REL_DIM = 16
N_REL = 1000
N_REL_PAD = 1024

NC = 2
NS = 16
NW = NC * NS
L = 16

B_TOTAL = E // NW
PHASES = 5
B_PHASE = B_TOTAL // PHASES
CHUNK = 96
FULL_CHUNKS = B_PHASE // CHUNK
TAIL_E = B_PHASE - FULL_CHUNKS * CHUNK


def _project_body(t_ref, w_ref, bias_ref, o_ref):
    wr = w_ref[EMB_DIM:EMB_DIM + REL_DIM, :]
    proj = jnp.dot(t_ref[...], wr, preferred_element_type=jnp.float32)
    o_ref[...] = proj + bias_ref[...]


def _project_table(rel_table, W, bias2d):
    in_dim = 2 * EMB_DIM + REL_DIM
    return pl.pallas_call(
        _project_body,
        in_specs=[
            pl.BlockSpec((N_REL, REL_DIM), lambda: (0, 0)),
            pl.BlockSpec((in_dim, EMB_DIM), lambda: (0, 0)),
            pl.BlockSpec((1, EMB_DIM), lambda: (0, 0)),
        ],
        out_specs=pl.BlockSpec((N_REL, EMB_DIM), lambda: (0, 0)),
        out_shape=jax.ShapeDtypeStruct((N_REL, EMB_DIM), jnp.float32),
    )(rel_table, W, bias2d)


_BCAST_DNUMS = lax.GatherDimensionNumbers(
    offset_dims=(), collapsed_slice_dims=(0,), start_index_map=(0,))


def _bcast_lane(vec, j):
    idx = jnp.full((L, 1), j, jnp.int32)
    return lax.gather(vec, idx, _BCAST_DNUMS, (1,),
                      mode=lax.GatherScatterMode.PROMISE_IN_BOUNDS)


def _sc_gather(table_pad, rel_ids):
    mesh = plsc.VectorSubcoreMesh(core_axis_name="c", subcore_axis_name="s")

    @functools.partial(
        pl.kernel,
        mesh=mesh,
        out_type=jax.ShapeDtypeStruct((E, EMB_DIM), jnp.float32),
        compiler_params=pltpu.CompilerParams(needs_layout_passes=False),
        scratch_types=[
            pltpu.VMEM((N_REL * EMB_DIM,), jnp.float32),
            pltpu.VMEM((B_PHASE,), jnp.int32),
            pltpu.VMEM((CHUNK, EMB_DIM), jnp.float32),
            pltpu.VMEM((CHUNK, EMB_DIM), jnp.float32),
            pltpu.SemaphoreType.DMA,
            pltpu.SemaphoreType.DMA,
        ],
    )
    def k(table_hbm, idx_hbm, out_hbm, table_v, idx_v, stg0, stg1, s0, s1):
        wid = lax.axis_index("s") * NC + lax.axis_index("c")
        base = wid * B_TOTAL
        pltpu.sync_copy(table_hbm, table_v)
        lanes = lax.iota(jnp.int32, L)
        zero16 = lanes * 0

        def gather_edge(stg_ref, goff, row):
            ids16 = plsc.load_gather(idx_v, [zero16 + goff])
            tbase = ids16 * EMB_DIM
            for kq in range(EMB_DIM // L):
                v = plsc.load_gather(table_v, [tbase + (lanes + kq * L)])
                stg_ref[row, pl.ds(kq * L, L)] = v

        def compute_chunk(stg_ref, c):
            @plsc.parallel_loop(0, CHUNK // L, unroll=2)
            def _(gg):
                gvec = idx_v[pl.ds(c * CHUNK + gg * L, L)]
                for j in range(L):
                    idsj = _bcast_lane(gvec, j)
                    tbase = idsj * EMB_DIM
                    for kq in range(EMB_DIM // L):
                        v = plsc.load_gather(
                            table_v, [tbase + (lanes + kq * L)])
                        stg_ref[gg * L + j, pl.ds(kq * L, L)] = v

        toff = FULL_CHUNKS * CHUNK

        @pl.loop(0, PHASES)
        def _(p):
            poff = base + p * B_PHASE
            pltpu.sync_copy(idx_hbm.at[pl.ds(poff, B_PHASE)], idx_v)

            def start(stg_ref, c, sem):
                pltpu.async_copy(
                    stg_ref, out_hbm.at[pl.ds(poff + c * CHUNK, CHUNK)], sem)

            def wait(stg_ref, sem):
                pltpu.make_async_copy(
                    stg_ref, out_hbm.at[pl.ds(poff, CHUNK)], sem).wait()

            compute_chunk(stg0, 0)
            start(stg0, 0, s0)
            compute_chunk(stg1, 1)
            start(stg1, 1, s1)

            @pl.loop(1, FULL_CHUNKS // 2)
            def _(t):
                wait(stg0, s0)
                compute_chunk(stg0, 2 * t)
                start(stg0, 2 * t, s0)
                wait(stg1, s1)
                compute_chunk(stg1, 2 * t + 1)
                start(stg1, 2 * t + 1, s1)

            wait(stg0, s0)

            @plsc.parallel_loop(0, TAIL_E, unroll=8)
            def _(e):
                gather_edge(stg0, toff + e, e)

            tail_src = stg0.at[pl.ds(0, TAIL_E)]
            tail_dst = out_hbm.at[pl.ds(poff + toff, TAIL_E)]
            pltpu.async_copy(tail_src, tail_dst, s0)
            pltpu.make_async_copy(tail_src, tail_dst, s0).wait()
            wait(stg1, s1)

    return k(table_pad, rel_ids)


def _tc_fused(emb_a, rel_g, emb_b, W):
    in_dim = 2 * EMB_DIM + REL_DIM
    return pl.pallas_call(
        _tc_body,
        grid=(E // BE,),
        in_specs=[
            pl.BlockSpec((BE, EMB_DIM), lambda i: (i, 0)),
            pl.BlockSpec((BE, EMB_DIM), lambda i: (i, 0)),
            pl.BlockSpec((BE, EMB_DIM), lambda i: (i, 0)),
            pl.BlockSpec((in_dim, EMB_DIM), lambda i: (0, 0)),
        ],
        out_specs=pl.BlockSpec((BE, EMB_DIM), lambda i: (i, 0)),
        out_shape=jax.ShapeDtypeStruct((E, EMB_DIM), jnp.float32),
        compiler_params=pltpu.CompilerParams(
            dimension_semantics=("parallel",),
        ),
    )(emb_a, rel_g, emb_b, W)


BE = 8000


def _tc_body(a_ref, rel_ref, b2_ref, w_ref, o_ref):
    wa = w_ref[0:EMB_DIM, :]
    wb = w_ref[EMB_DIM + REL_DIM:, :]
    acc = jnp.dot(a_ref[...], wa, preferred_element_type=jnp.float32)
    acc += jnp.dot(b2_ref[...], wb, preferred_element_type=jnp.float32)
    o_ref[...] = acc + rel_ref[...]


def kernel(emb_a, rel_ids, emb_b, rel_table, W, b):
    table_pad = _project_table(rel_table, W, b.reshape(1, EMB_DIM))
    table_flat = table_pad.reshape(N_REL * EMB_DIM)
    rel_g = _sc_gather(table_flat, rel_ids)
    return _tc_fused(emb_a, rel_g, emb_b, W)


# speedup vs baseline: 1.0303x; 1.0014x over previous
import functools

import jax
import jax.numpy as jnp
from jax import lax
from jax.experimental import pallas as pl
from jax.experimental.pallas import tpu as pltpu
from jax.experimental.pallas import tpu_sc as plsc

E = 800000
EMB_DIM = 64
REL_DIM = 16
N_REL = 1000
N_REL_PAD = 1024
PACK = EMB_DIM // 2

NC = 2
NS = 16
NW = NC * NS
L = 16

B_TOTAL = E // NW
PHASES = 5
B_PHASE = B_TOTAL // PHASES
CHUNK = 192
FULL_CHUNKS = B_PHASE // CHUNK
TAIL_E = B_PHASE - FULL_CHUNKS * CHUNK


def _project_body(t_ref, w_ref, bias_ref, o_ref):
    wr = w_ref[EMB_DIM:EMB_DIM + REL_DIM, :]
    proj = jnp.dot(t_ref[...], wr, preferred_element_type=jnp.float32)
    o_ref[...] = proj + bias_ref[...]


def _project_table(rel_table, W, bias2d):
    in_dim = 2 * EMB_DIM + REL_DIM
    return pl.pallas_call(
        _project_body,
        in_specs=[
            pl.BlockSpec((N_REL, REL_DIM), lambda: (0, 0)),
            pl.BlockSpec((in_dim, EMB_DIM), lambda: (0, 0)),
            pl.BlockSpec((1, EMB_DIM), lambda: (0, 0)),
        ],
        out_specs=pl.BlockSpec((N_REL, EMB_DIM), lambda: (0, 0)),
        out_shape=jax.ShapeDtypeStruct((N_REL, EMB_DIM), jnp.float32),
    )(rel_table, W, bias2d)


_BCAST_DNUMS = lax.GatherDimensionNumbers(
    offset_dims=(), collapsed_slice_dims=(0,), start_index_map=(0,))


def _bcast_lane(vec, j):
    idx = jnp.full((L, 1), j, jnp.int32)
    return lax.gather(vec, idx, _BCAST_DNUMS, (1,),
                      mode=lax.GatherScatterMode.PROMISE_IN_BOUNDS)


def _sc_gather(table_pad, rel_ids):
    mesh = plsc.VectorSubcoreMesh(core_axis_name="c", subcore_axis_name="s")

    @functools.partial(
        pl.kernel,
        mesh=mesh,
        out_type=jax.ShapeDtypeStruct((E, PACK), jnp.int32),
        compiler_params=pltpu.CompilerParams(needs_layout_passes=False),
        scratch_types=[
            pltpu.VMEM((N_REL * PACK,), jnp.int32),
            pltpu.VMEM((B_PHASE,), jnp.int32),
            pltpu.VMEM((CHUNK, PACK), jnp.int32),
            pltpu.VMEM((CHUNK, PACK), jnp.int32),
            pltpu.SemaphoreType.DMA,
            pltpu.SemaphoreType.DMA,
        ],
    )
    def k(table_hbm, idx_hbm, out_hbm, table_v, idx_v, stg0, stg1, s0, s1):
        wid = lax.axis_index("s") * NC + lax.axis_index("c")
        base = wid * B_TOTAL
        pltpu.sync_copy(table_hbm, table_v)
        lanes = lax.iota(jnp.int32, L)
        zero16 = lanes * 0

        def gather_edge(stg_ref, goff, row):
            ids16 = plsc.load_gather(idx_v, [zero16 + goff])
            tbase = ids16 * PACK
            for kq in range(PACK // L):
                v = plsc.load_gather(table_v, [tbase + (lanes + kq * L)])
                stg_ref[row, pl.ds(kq * L, L)] = v

        def compute_chunk(stg_ref, c):
            @plsc.parallel_loop(0, CHUNK // L, unroll=2)
            def _(gg):
                gvec = idx_v[pl.ds(c * CHUNK + gg * L, L)]
                for j in range(L):
                    idsj = _bcast_lane(gvec, j)
                    tbase = idsj * PACK
                    for kq in range(PACK // L):
                        v = plsc.load_gather(
                            table_v, [tbase + (lanes + kq * L)])
                        stg_ref[gg * L + j, pl.ds(kq * L, L)] = v

        toff = FULL_CHUNKS * CHUNK

        @pl.loop(0, PHASES)
        def _(p):
            poff = base + p * B_PHASE
            pltpu.sync_copy(idx_hbm.at[pl.ds(poff, B_PHASE)], idx_v)

            def start(stg_ref, c, sem):
                pltpu.async_copy(
                    stg_ref, out_hbm.at[pl.ds(poff + c * CHUNK, CHUNK)], sem)

            def wait(stg_ref, sem):
                pltpu.make_async_copy(
                    stg_ref, out_hbm.at[pl.ds(poff, CHUNK)], sem).wait()

            compute_chunk(stg0, 0)
            start(stg0, 0, s0)
            compute_chunk(stg1, 1)
            start(stg1, 1, s1)

            @pl.loop(1, FULL_CHUNKS // 2)
            def _(t):
                wait(stg0, s0)
                compute_chunk(stg0, 2 * t)
                start(stg0, 2 * t, s0)
                wait(stg1, s1)
                compute_chunk(stg1, 2 * t + 1)
                start(stg1, 2 * t + 1, s1)

            wait(stg0, s0)

            @plsc.parallel_loop(0, TAIL_E, unroll=8)
            def _(e):
                gather_edge(stg0, toff + e, e)

            tail_src = stg0.at[pl.ds(0, TAIL_E)]
            tail_dst = out_hbm.at[pl.ds(poff + toff, TAIL_E)]
            pltpu.async_copy(tail_src, tail_dst, s0)
            pltpu.make_async_copy(tail_src, tail_dst, s0).wait()
            wait(stg1, s1)

    return k(table_pad, rel_ids)


def _tc_fused(emb_a, rel_g, emb_b, W):
    in_dim = 2 * EMB_DIM + REL_DIM
    return pl.pallas_call(
        _tc_body,
        grid=(E // BE,),
        in_specs=[
            pl.BlockSpec((BE, EMB_DIM), lambda i: (i, 0)),
            pl.BlockSpec((BE, PACK), lambda i: (i, 0)),
            pl.BlockSpec((BE, EMB_DIM), lambda i: (i, 0)),
            pl.BlockSpec((in_dim, EMB_DIM), lambda i: (0, 0)),
        ],
        out_specs=pl.BlockSpec((BE, EMB_DIM), lambda i: (i, 0)),
        out_shape=jax.ShapeDtypeStruct((E, EMB_DIM), jnp.float32),
        compiler_params=pltpu.CompilerParams(
            dimension_semantics=("parallel",),
        ),
    )(emb_a, rel_g, emb_b, W)


BE = 8000


def _tc_body(a_ref, rel_ref, b2_ref, w_ref, o_ref):
    wa = w_ref[0:EMB_DIM, :]
    wb = w_ref[EMB_DIM + REL_DIM:, :]
    acc = jnp.dot(a_ref[...], wa, preferred_element_type=jnp.float32)
    acc += jnp.dot(b2_ref[...], wb, preferred_element_type=jnp.float32)
    x = rel_ref[...]
    lo = lax.bitcast_convert_type(x << 16, jnp.float32)
    hi = lax.bitcast_convert_type(x & jnp.int32(-65536), jnp.float32)
    o_ref[...] = acc + jnp.concatenate([lo, hi], axis=-1)


def kernel(emb_a, rel_ids, emb_b, rel_table, W, b):
    proj = _project_table(rel_table, W, b.reshape(1, EMB_DIM))
    lo = lax.bitcast_convert_type(
        proj[:, :PACK].astype(jnp.bfloat16), jnp.uint16).astype(jnp.uint32)
    hi = lax.bitcast_convert_type(
        proj[:, PACK:].astype(jnp.bfloat16), jnp.uint16).astype(jnp.uint32)
    table_pack = lax.bitcast_convert_type(lo | (hi << 16), jnp.int32)
    rel_g = _sc_gather(table_pack.reshape(N_REL * PACK), rel_ids)
    return _tc_fused(emb_a, rel_g, emb_b, W)


# speedup vs baseline: 2.9886x; 2.9007x over previous
import functools

import jax
import jax.numpy as jnp
from jax import lax
from jax.experimental import pallas as pl
from jax.experimental.pallas import tpu as pltpu
from jax.experimental.pallas import tpu_sc as plsc

E = 800000
EMB_DIM = 64
REL_DIM = 16
N_REL = 1000
N_REL_PAD = 1024
PACK = EMB_DIM // 2

NC = 2
NS = 16
NW = NC * NS
L = 16

B_TOTAL = E // NW
PHASES = 5
B_PHASE = B_TOTAL // PHASES
CHUNK = 192
FULL_CHUNKS = B_PHASE // CHUNK
TAIL_E = B_PHASE - FULL_CHUNKS * CHUNK


def _project_body(t_ref, w_ref, bias_ref, o_ref):
    wr = w_ref[EMB_DIM:EMB_DIM + REL_DIM, :]
    proj = jnp.dot(t_ref[...], wr, preferred_element_type=jnp.float32)
    o_ref[...] = proj + bias_ref[...]


def _project_table(rel_table, W, bias2d):
    in_dim = 2 * EMB_DIM + REL_DIM
    return pl.pallas_call(
        _project_body,
        in_specs=[
            pl.BlockSpec((N_REL, REL_DIM), lambda: (0, 0)),
            pl.BlockSpec((in_dim, EMB_DIM), lambda: (0, 0)),
            pl.BlockSpec((1, EMB_DIM), lambda: (0, 0)),
        ],
        out_specs=pl.BlockSpec((N_REL, EMB_DIM), lambda: (0, 0)),
        out_shape=jax.ShapeDtypeStruct((N_REL, EMB_DIM), jnp.float32),
    )(rel_table, W, bias2d)


_BCAST_DNUMS = lax.GatherDimensionNumbers(
    offset_dims=(), collapsed_slice_dims=(0,), start_index_map=(0,))


def _bcast_lane(vec, j):
    idx = jnp.full((L, 1), j, jnp.int32)
    return lax.gather(vec, idx, _BCAST_DNUMS, (1,),
                      mode=lax.GatherScatterMode.PROMISE_IN_BOUNDS)


def _sc_gather(table_pad, rel_ids):
    mesh = plsc.VectorSubcoreMesh(core_axis_name="c", subcore_axis_name="s")

    @functools.partial(
        pl.kernel,
        mesh=mesh,
        out_type=jax.ShapeDtypeStruct((E, PACK), jnp.int32),
        compiler_params=pltpu.CompilerParams(needs_layout_passes=False),
        scratch_types=[
            pltpu.VMEM((N_REL * PACK,), jnp.int32),
            pltpu.VMEM((B_PHASE,), jnp.int32),
            pltpu.VMEM((CHUNK, PACK), jnp.int32),
            pltpu.VMEM((CHUNK, PACK), jnp.int32),
            pltpu.SemaphoreType.DMA,
            pltpu.SemaphoreType.DMA,
        ],
    )
    def k(table_hbm, idx_hbm, out_hbm, table_v, idx_v, stg0, stg1, s0, s1):
        wid = lax.axis_index("s") * NC + lax.axis_index("c")
        base = wid * B_TOTAL
        pltpu.sync_copy(table_hbm, table_v)
        lanes = lax.iota(jnp.int32, L)
        zero16 = lanes * 0

        def gather_edge(stg_ref, goff, row):
            ids16 = plsc.load_gather(idx_v, [zero16 + goff])
            tbase = ids16 * PACK
            for kq in range(PACK // L):
                v = plsc.load_gather(table_v, [tbase + (lanes + kq * L)])
                stg_ref[row, pl.ds(kq * L, L)] = v

        def compute_chunk(stg_ref, c):
            @plsc.parallel_loop(0, CHUNK // L, unroll=2)
            def _(gg):
                gvec = idx_v[pl.ds(c * CHUNK + gg * L, L)]
                for j in range(L):
                    idsj = _bcast_lane(gvec, j)
                    tbase = idsj * PACK
                    for kq in range(PACK // L):
                        v = plsc.load_gather(
                            table_v, [tbase + (lanes + kq * L)])
                        stg_ref[gg * L + j, pl.ds(kq * L, L)] = v

        toff = FULL_CHUNKS * CHUNK

        @pl.loop(0, PHASES)
        def _(p):
            poff = base + p * B_PHASE
            pltpu.sync_copy(idx_hbm.at[pl.ds(poff, B_PHASE)], idx_v)

            def start(stg_ref, c, sem):
                pltpu.async_copy(
                    stg_ref, out_hbm.at[pl.ds(poff + c * CHUNK, CHUNK)], sem)

            def wait(stg_ref, sem):
                pltpu.make_async_copy(
                    stg_ref, out_hbm.at[pl.ds(poff, CHUNK)], sem).wait()

            compute_chunk(stg0, 0)
            start(stg0, 0, s0)
            compute_chunk(stg1, 1)
            start(stg1, 1, s1)

            @pl.loop(1, FULL_CHUNKS // 2)
            def _(t):
                wait(stg0, s0)
                compute_chunk(stg0, 2 * t)
                start(stg0, 2 * t, s0)
                wait(stg1, s1)
                compute_chunk(stg1, 2 * t + 1)
                start(stg1, 2 * t + 1, s1)

            wait(stg0, s0)

            @plsc.parallel_loop(0, TAIL_E, unroll=8)
            def _(e):
                gather_edge(stg0, toff + e, e)

            tail_src = stg0.at[pl.ds(0, TAIL_E)]
            tail_dst = out_hbm.at[pl.ds(poff + toff, TAIL_E)]
            pltpu.async_copy(tail_src, tail_dst, s0)
            pltpu.make_async_copy(tail_src, tail_dst, s0).wait()
            wait(stg1, s1)

    return k(table_pad, rel_ids)


def _tc_fused(embT_a, rel_g, embT_b, WT):
    in_dim = 2 * EMB_DIM + REL_DIM
    return pl.pallas_call(
        _tc_body,
        grid=(E // BE,),
        in_specs=[
            pl.BlockSpec((EMB_DIM, BE), lambda i: (0, i)),
            pl.BlockSpec((BE, PACK), lambda i: (i, 0)),
            pl.BlockSpec((EMB_DIM, BE), lambda i: (0, i)),
            pl.BlockSpec((EMB_DIM, in_dim), lambda i: (0, 0)),
        ],
        out_specs=pl.BlockSpec((EMB_DIM, BE), lambda i: (0, i)),
        out_shape=jax.ShapeDtypeStruct((EMB_DIM, E), jnp.float32),
        compiler_params=pltpu.CompilerParams(
            dimension_semantics=("parallel",),
        ),
    )(embT_a, rel_g, embT_b, WT)


BE = 6400


def _tc_body(aT_ref, rel_ref, bT_ref, wt_ref, o_ref):
    waT = wt_ref[:, 0:EMB_DIM]
    wbT = wt_ref[:, EMB_DIM + REL_DIM:]
    accT = jnp.dot(waT, aT_ref[...], preferred_element_type=jnp.float32)
    accT += jnp.dot(wbT, bT_ref[...], preferred_element_type=jnp.float32)
    x = rel_ref[...]
    lo = lax.bitcast_convert_type(x << 16, jnp.float32)
    hi = lax.bitcast_convert_type(x & jnp.int32(-65536), jnp.float32)
    rel = jnp.concatenate([lo, hi], axis=-1)
    r = lax.broadcasted_iota(jnp.int32, (EMB_DIM, EMB_DIM), 0)
    c = lax.broadcasted_iota(jnp.int32, (EMB_DIM, EMB_DIM), 1)
    eye = jnp.where(r == c, 1.0, 0.0).astype(jnp.float32)
    relT = lax.dot_general(eye, rel, (((1,), (1,)), ((), ())),
                           preferred_element_type=jnp.float32)
    o_ref[...] = accT + relT


def kernel(emb_a, rel_ids, emb_b, rel_table, W, b):
    proj = _project_table(rel_table, W, b.reshape(1, EMB_DIM))
    lo = lax.bitcast_convert_type(
        proj[:, :PACK].astype(jnp.bfloat16), jnp.uint16).astype(jnp.uint32)
    hi = lax.bitcast_convert_type(
        proj[:, PACK:].astype(jnp.bfloat16), jnp.uint16).astype(jnp.uint32)
    table_pack = lax.bitcast_convert_type(lo | (hi << 16), jnp.int32)
    rel_g = _sc_gather(table_pack.reshape(N_REL * PACK), rel_ids)
    outT = _tc_fused(emb_a.T, rel_g, emb_b.T, W.T)
    return outT.T
